# baseline (speedup 1.0000x reference)
import functools

import jax
import jax.numpy as jnp
from jax import lax
from jax.experimental import pallas as pl
from jax.experimental.pallas import tpu as pltpu
from jax.experimental.pallas import tpu_sc as plsc

f32 = jnp.float32

_N = 10000
_E = 160000
_G = 32
_D = 32
_STEPS = 6
_S2S_ITERS = 6

_NC = 2
_NS = 16
_NW = _NC * _NS
_CHUNK = 128
_NCHUNK = 40
_W_EDGES = _NCHUNK * _CHUNK
_E_PAD = _NW * _W_EDGES
_N_SP = 10016
_ROWS_PER_SUB = _N_SP // _NS

_BLK_N = 2000
_BLK_E = 2048


def _sc_mesh():
    return plsc.VectorSubcoreMesh(core_axis_name="c", subcore_axis_name="s")


_NBUF = 8
_NGRP = _NCHUNK // _NBUF


@functools.partial(
    pl.kernel,
    mesh=_sc_mesh(),
    compiler_params=pltpu.CompilerParams(use_tc_tiling_on_sc=False),
    out_type=jax.ShapeDtypeStruct((_E_PAD, _D), f32),
    scratch_types=[
        pltpu.VMEM((_NCHUNK, _CHUNK), jnp.int32),
        pltpu.VMEM((_NBUF, _CHUNK, _D), f32),
        pltpu.VMEM_SHARED((_N, _D), f32),
        pltpu.SemaphoreType.DMA,
        pltpu.SemaphoreType.DMA,
    ],
)
def _sc_gather(x_hbm, src_hbm, out_hbm, idx_v, rows_v, x_sh, sem_g, sem_o):
    c = lax.axis_index("c")
    s = lax.axis_index("s")
    wid = c * _NS + s
    rows_stage = _N // _NS
    pltpu.sync_copy(x_hbm.at[pl.ds(s * rows_stage, rows_stage)],
                    x_sh.at[pl.ds(s * rows_stage, rows_stage)])
    pltpu.sync_copy(src_hbm.at[wid], idx_v)
    plsc.subcore_barrier()
    base = wid * _W_EDGES

    def group(g, carry):
        @pl.when(g > 0)
        def _drain_prev_out():
            for b in range(_NBUF):
                off = base + ((g - 1) * _NBUF + b) * _CHUNK
                pltpu.make_async_copy(
                    rows_v.at[b], out_hbm.at[pl.ds(off, _CHUNK)], sem_o).wait()
        for b in range(_NBUF):
            pltpu.async_copy(x_sh.at[idx_v.at[g * _NBUF + b]], rows_v.at[b], sem_g)
        for b in range(_NBUF):
            pltpu.make_async_copy(
                x_sh.at[idx_v.at[g * _NBUF + b]], rows_v.at[b], sem_g).wait()
        for b in range(_NBUF):
            off = base + (g * _NBUF + b) * _CHUNK
            pltpu.async_copy(rows_v.at[b], out_hbm.at[pl.ds(off, _CHUNK)], sem_o)
        return carry

    lax.fori_loop(0, _NGRP, group, 0)
    for b in range(_NBUF):
        off = base + ((_NGRP - 1) * _NBUF + b) * _CHUNK
        pltpu.make_async_copy(
            rows_v.at[b], out_hbm.at[pl.ds(off, _CHUNK)], sem_o).wait()


@functools.partial(
    pl.kernel,
    mesh=_sc_mesh(),
    compiler_params=pltpu.CompilerParams(use_tc_tiling_on_sc=False),
    out_type=jax.ShapeDtypeStruct((_NC, _N_SP, _D), f32),
    scratch_types=[
        pltpu.VMEM((_NCHUNK, _CHUNK), jnp.int32),
        pltpu.VMEM((_NBUF, _CHUNK, _D), f32),
        pltpu.VMEM_SHARED((_N_SP, _D), f32),
        pltpu.SemaphoreType.DMA,
        pltpu.SemaphoreType.DMA,
    ],
)
def _sc_scatter(msg_hbm, dst_hbm, zero_hbm, out_hbm, idx_v, rows_v, acc_sh,
                sem_l, sem_s):
    c = lax.axis_index("c")
    s = lax.axis_index("s")
    wid = c * _NS + s

    @pl.when(s == 0)
    def _init():
        pltpu.sync_copy(zero_hbm, acc_sh)

    plsc.subcore_barrier()
    pltpu.sync_copy(dst_hbm.at[wid], idx_v)
    base = wid * _W_EDGES

    def group(g, carry):
        @pl.when(g > 0)
        def _drain_prev_adds():
            for b in range(_NBUF):
                j = (g - 1) * _NBUF + b
                pltpu.make_async_copy(
                    rows_v.at[b], acc_sh.at[idx_v.at[j]], sem_s).wait()
        for b in range(_NBUF):
            j = g * _NBUF + b
            pltpu.async_copy(
                msg_hbm.at[pl.ds(base + j * _CHUNK, _CHUNK)], rows_v.at[b], sem_l)
        for b in range(_NBUF):
            j = g * _NBUF + b
            pltpu.make_async_copy(
                msg_hbm.at[pl.ds(base + j * _CHUNK, _CHUNK)], rows_v.at[b],
                sem_l).wait()
        for b in range(_NBUF):
            j = g * _NBUF + b
            pltpu.async_copy(rows_v.at[b], acc_sh.at[idx_v.at[j]], sem_s, add=True)
        return carry

    lax.fori_loop(0, _NGRP, group, 0)
    for b in range(_NBUF):
        j = (_NGRP - 1) * _NBUF + b
        pltpu.make_async_copy(rows_v.at[b], acc_sh.at[idx_v.at[j]], sem_s).wait()
    plsc.subcore_barrier()
    r0 = s * _ROWS_PER_SUB
    pltpu.sync_copy(acc_sh.at[pl.ds(r0, _ROWS_PER_SUB)],
                    out_hbm.at[c, pl.ds(r0, _ROWS_PER_SUB)])


_E_HALF = _E_PAD // 2
_NCHUNK_H = 20
_W_EDGES_H = _NCHUNK_H * _CHUNK
_NBUF_H = 5
_NGRP_H = _NCHUNK_H // _NBUF_H


@functools.partial(
    pl.kernel,
    mesh=_sc_mesh(),
    compiler_params=pltpu.CompilerParams(use_tc_tiling_on_sc=False),
    out_type=jax.ShapeDtypeStruct((_E_HALF, _D), f32),
    scratch_types=[
        pltpu.VMEM((_NCHUNK_H, _CHUNK), jnp.int32),
        pltpu.VMEM((_NBUF_H, _CHUNK, _D), f32),
        pltpu.VMEM_SHARED((_N, _D), f32),
        pltpu.SemaphoreType.DMA,
        pltpu.SemaphoreType.DMA,
    ],
)
def _sc_gather_h(x_hbm, src_hbm, out_hbm, idx_v, rows_v, x_sh, sem_g, sem_o):
    c = lax.axis_index("c")
    s = lax.axis_index("s")
    wid = c * _NS + s
    rows_stage = _N // _NS
    pltpu.sync_copy(x_hbm.at[pl.ds(s * rows_stage, rows_stage)],
                    x_sh.at[pl.ds(s * rows_stage, rows_stage)])
    pltpu.sync_copy(src_hbm.at[wid], idx_v)
    plsc.subcore_barrier()
    base = wid * _W_EDGES_H

    def group(g, carry):
        @pl.when(g > 0)
        def _drain_prev_out():
            for b in range(_NBUF_H):
                off = base + ((g - 1) * _NBUF_H + b) * _CHUNK
                pltpu.make_async_copy(
                    rows_v.at[b], out_hbm.at[pl.ds(off, _CHUNK)], sem_o).wait()
        for b in range(_NBUF_H):
            pltpu.async_copy(x_sh.at[idx_v.at[g * _NBUF_H + b]], rows_v.at[b],
                             sem_g)
        for b in range(_NBUF_H):
            pltpu.make_async_copy(
                x_sh.at[idx_v.at[g * _NBUF_H + b]], rows_v.at[b], sem_g).wait()
        for b in range(_NBUF_H):
            off = base + (g * _NBUF_H + b) * _CHUNK
            pltpu.async_copy(rows_v.at[b], out_hbm.at[pl.ds(off, _CHUNK)], sem_o)
        return carry

    lax.fori_loop(0, _NGRP_H, group, 0)
    for b in range(_NBUF_H):
        off = base + ((_NGRP_H - 1) * _NBUF_H + b) * _CHUNK
        pltpu.make_async_copy(
            rows_v.at[b], out_hbm.at[pl.ds(off, _CHUNK)], sem_o).wait()


@functools.partial(
    pl.kernel,
    mesh=_sc_mesh(),
    compiler_params=pltpu.CompilerParams(use_tc_tiling_on_sc=False),
    out_type=jax.ShapeDtypeStruct((_NC, _N_SP, _D), f32),
    scratch_types=[
        pltpu.VMEM((_NCHUNK_H, _CHUNK), jnp.int32),
        pltpu.VMEM((_NBUF_H, _CHUNK, _D), f32),
        pltpu.VMEM_SHARED((_N_SP, _D), f32),
        pltpu.SemaphoreType.DMA,
        pltpu.SemaphoreType.DMA,
    ],
)
def _sc_scatter_h(msg_hbm, dst_hbm, zero_hbm, out_hbm, idx_v, rows_v, acc_sh,
                  sem_l, sem_s):
    c = lax.axis_index("c")
    s = lax.axis_index("s")
    wid = c * _NS + s

    @pl.when(s == 0)
    def _init():
        pltpu.sync_copy(zero_hbm, acc_sh)

    plsc.subcore_barrier()
    pltpu.sync_copy(dst_hbm.at[wid], idx_v)
    base = wid * _W_EDGES_H

    def group(g, carry):
        @pl.when(g > 0)
        def _drain_prev_adds():
            for b in range(_NBUF_H):
                j = (g - 1) * _NBUF_H + b
                pltpu.make_async_copy(
                    rows_v.at[b], acc_sh.at[idx_v.at[j]], sem_s).wait()
        for b in range(_NBUF_H):
            j = g * _NBUF_H + b
            pltpu.async_copy(
                msg_hbm.at[pl.ds(base + j * _CHUNK, _CHUNK)], rows_v.at[b], sem_l)
        for b in range(_NBUF_H):
            j = g * _NBUF_H + b
            pltpu.make_async_copy(
                msg_hbm.at[pl.ds(base + j * _CHUNK, _CHUNK)], rows_v.at[b],
                sem_l).wait()
        for b in range(_NBUF_H):
            j = g * _NBUF_H + b
            pltpu.async_copy(rows_v.at[b], acc_sh.at[idx_v.at[j]], sem_s,
                             add=True)
        return carry

    lax.fori_loop(0, _NGRP_H, group, 0)
    for b in range(_NBUF_H):
        j = (_NGRP_H - 1) * _NBUF_H + b
        pltpu.make_async_copy(rows_v.at[b], acc_sh.at[idx_v.at[j]], sem_s).wait()
    plsc.subcore_barrier()
    r0 = s * _ROWS_PER_SUB
    pltpu.sync_copy(acc_sh.at[pl.ds(r0, _ROWS_PER_SUB)],
                    out_hbm.at[c, pl.ds(r0, _ROWS_PER_SUB)])


def _full(shape):
    return pl.BlockSpec(shape, lambda *_: (0,) * len(shape))


def _node_feat_body(pu, pd, nf, sd, emb, w, b, o_ref):
    nfv = nf[...]
    oh = (nfv == lax.broadcasted_iota(jnp.int32, (_BLK_N, 9), 1).astype(f32)).astype(f32)
    nf_emb = jnp.dot(oh, emb[...], preferred_element_type=f32)
    wv = w[...]
    x = (jnp.dot(pu[...], wv[0:16], preferred_element_type=f32)
         + jnp.dot(pd[...], wv[16:32], preferred_element_type=f32)
         + jnp.dot(nf_emb, wv[32:64], preferred_element_type=f32)
         + sd[...] * wv[64:65] + (nfv / 8.0) * wv[65:66] + b[...])
    o_ref[...] = jnp.maximum(x, 0.0)


def _node_feat(pu, pd, nf, sd, emb, w, b):
    grid = (_N // _BLK_N,)
    return pl.pallas_call(
        _node_feat_body,
        grid=grid,
        in_specs=[
            pl.BlockSpec((_BLK_N, 16), lambda i: (i, 0)),
            pl.BlockSpec((_BLK_N, 16), lambda i: (i, 0)),
            pl.BlockSpec((_BLK_N, 1), lambda i: (i, 0)),
            pl.BlockSpec((_BLK_N, 1), lambda i: (i, 0)),
            _full((9, 32)), _full((66, 32)), _full((1, 32)),
        ],
        out_specs=pl.BlockSpec((_BLK_N, _D), lambda i: (i, 0)),
        out_shape=jax.ShapeDtypeStruct((_N, _D), f32),
    )(pu, pd, nf, sd, emb, w, b)


def _edge_a_body(ef, emb, w1a, w1l, b1, o_ref):
    efv = ef[...]
    oh = (efv == lax.broadcasted_iota(jnp.int32, (_BLK_E, 9), 1).astype(f32)).astype(f32)
    ef_emb = jnp.dot(oh, emb[...], preferred_element_type=f32)
    o_ref[...] = jnp.maximum(
        jnp.dot(ef_emb, w1a[...], preferred_element_type=f32)
        + (efv / 8.0) * w1l[...] + b1[...], 0.0)


def _edge_a(ef, emb, w1a, w1l, b1):
    grid = (_E_PAD // _BLK_E,)
    return pl.pallas_call(
        _edge_a_body,
        grid=grid,
        in_specs=[
            pl.BlockSpec((_BLK_E, 1), lambda i: (i, 0)),
            _full((9, 32)), _full((32, 32)), _full((1, 32)), _full((1, 32)),
        ],
        out_specs=pl.BlockSpec((_BLK_E, 32), lambda i: (i, 0)),
        out_shape=jax.ShapeDtypeStruct((_E_PAD, 32), f32),
    )(ef, emb, w1a, w1l, b1)


def _msg_body(xs, a, w2r, bmat, o_ref):
    bf = jnp.bfloat16
    xv = xs[...]
    av = a[...]
    ci = lax.broadcasted_iota(jnp.int32, (32, 1024), 1)
    rk = lax.broadcasted_iota(jnp.int32, (32, 1024), 0)
    rep = (ci // 32 == rk).astype(bf)
    arep = jnp.dot(av.astype(bf), rep, preferred_element_type=f32)
    xtile = pltpu.repeat(xv, 32, axis=1)
    u = (arep * xtile).astype(bf)
    o_ref[...] = (jnp.dot(u, w2r[...], preferred_element_type=f32)
                  + jnp.dot(xv, bmat[...], preferred_element_type=f32))


def _msg(xs, a, w2r, bmat, epad=_E_PAD):
    grid = (epad // _BLK_E,)
    return pl.pallas_call(
        _msg_body,
        grid=grid,
        in_specs=[
            pl.BlockSpec((_BLK_E, _D), lambda i: (i, 0)),
            pl.BlockSpec((_BLK_E, _D), lambda i: (i, 0)),
            _full((1024, 32)), _full((32, 32)),
        ],
        out_specs=pl.BlockSpec((_BLK_E, _D), lambda i: (i, 0)),
        out_shape=jax.ShapeDtypeStruct((epad, _D), f32),
    )(xs, a, w2r, bmat)


def _gru_body(p0, p1, p2, p3, h, cb, wir, wiz, win, whr, whz, whn,
              bir, biz, bin_, bhr, bhz, bhn, o_ref):
    agg = (p0[...] + p1[...] + p2[...] + p3[...])[0]
    m = jnp.maximum(agg + cb[...], 0.0)
    hv = h[...]
    dot = lambda a, b: jnp.dot(a, b, preferred_element_type=f32)
    r = jax.nn.sigmoid(dot(m, wir[...]) + bir[...] + dot(hv, whr[...]) + bhr[...])
    z = jax.nn.sigmoid(dot(m, wiz[...]) + biz[...] + dot(hv, whz[...]) + bhz[...])
    nn_ = jnp.tanh(dot(m, win[...]) + bin_[...] + r * (dot(hv, whn[...]) + bhn[...]))
    o_ref[...] = (1.0 - z) * nn_ + z * hv


def _gru(parts_a, parts_b, h, cb, ws):
    grid = (_N // _BLK_N,)
    specs = [pl.BlockSpec((1, _BLK_N, _D), lambda i: (0, i, 0)),
             pl.BlockSpec((1, _BLK_N, _D), lambda i: (1, i, 0)),
             pl.BlockSpec((1, _BLK_N, _D), lambda i: (0, i, 0)),
             pl.BlockSpec((1, _BLK_N, _D), lambda i: (1, i, 0)),
             pl.BlockSpec((_BLK_N, _D), lambda i: (i, 0))]
    specs += [_full((1, 32))]
    specs += [_full((32, 32))] * 6
    specs += [_full((1, 32))] * 6
    return pl.pallas_call(
        _gru_body,
        grid=grid,
        in_specs=specs,
        out_specs=pl.BlockSpec((_BLK_N, _D), lambda i: (i, 0)),
        out_shape=jax.ShapeDtypeStruct((_N, _D), f32),
    )(parts_a, parts_a, parts_b, parts_b, h, cb, *ws)


def _s2s_body(feat, featT, gidT, *refs):
    out_ref = refs[-1]
    flat = refs[:-1]
    wih_i, wih_f, wih_g, wih_o = [flat[3 * g:3 * g + 3] for g in range(4)]
    whh_i, whh_f, whh_g, whh_o = [flat[12 + 3 * g:12 + 3 * g + 3] for g in range(4)]
    b_i, b_f, b_g, b_o = [flat[24 + 3 * g:24 + 3 * g + 3] for g in range(4)]
    row1, rob1, row2, rob2 = flat[36:40]

    featv = feat[...]
    featTv = featT[...]
    mT = gidT[...] == lax.broadcasted_iota(jnp.int32, (_G, _N), 0).astype(f32)

    dot = lambda a, b: jnp.dot(a, b, preferred_element_type=f32)
    hs = [jnp.zeros((_G, _D), f32) for _ in range(3)]
    cs = [jnp.zeros((_G, _D), f32) for _ in range(3)]
    q_star = jnp.zeros((_G, 2 * _D), f32)
    for _ in range(_S2S_ITERS):
        x_in = q_star
        for l in range(3):
            pre_i = dot(x_in, wih_i[l][...]) + dot(hs[l], whh_i[l][...]) + b_i[l][...]
            pre_f = dot(x_in, wih_f[l][...]) + dot(hs[l], whh_f[l][...]) + b_f[l][...]
            pre_g = dot(x_in, wih_g[l][...]) + dot(hs[l], whh_g[l][...]) + b_g[l][...]
            pre_o = dot(x_in, wih_o[l][...]) + dot(hs[l], whh_o[l][...]) + b_o[l][...]
            ig = jax.nn.sigmoid(pre_i)
            fg = jax.nn.sigmoid(pre_f)
            gg = jnp.tanh(pre_g)
            og = jax.nn.sigmoid(pre_o)
            cs[l] = fg * cs[l] + ig * gg
            hs[l] = og * jnp.tanh(cs[l])
            x_in = hs[l]
        q = x_in
        e_allT = dot(q, featTv)
        emaxT = jnp.max(jnp.where(mT, e_allT, -1e30), axis=1, keepdims=True)
        exT = jnp.exp(jnp.where(mT, e_allT - emaxT, -1e30))
        esumT = jnp.sum(exT, axis=1, keepdims=True)
        alphaT = exT / jnp.maximum(esumT, 1e-30)
        readout = dot(alphaT, featv)
        q_star = jnp.concatenate([q, readout], axis=-1)
    hid = jnp.maximum(dot(q_star, row1[...]) + rob1[...], 0.0)
    out_ref[...] = dot(hid, row2[...]) + rob2[...]


def _s2s(feat, featT, gidT, lstm_parts, ro):
    args = [feat, featT, gidT] + list(lstm_parts) + list(ro)
    in_specs = [_full(a.shape) for a in args]
    return pl.pallas_call(
        _s2s_body,
        in_specs=in_specs,
        out_specs=_full((_G, 32)),
        out_shape=jax.ShapeDtypeStruct((_G, 32), f32),
        grid=(1,),
    )(*args)


def kernel(pos_undirected, pos_directed, nfreq, seed, efreq, edge_index, graph_ids,
           node_freq_emb, edge_freq_emb, lin0_W, lin0_b, en_W1, en_b1, en_W2, en_b2,
           conv_bias, gru_Wih, gru_Whh, gru_bih, gru_bhh,
           lstm_Wih0, lstm_Wih12, lstm_Whh, lstm_bih, lstm_bhh,
           ro_W1, ro_b1, ro_W2, ro_b2):
    nf = jnp.clip(nfreq, 0, 8).astype(f32)[:, None]
    sd = seed.astype(f32)[:, None]
    ef = jnp.clip(efreq, 0, 8).astype(f32)[:, None]
    ef_pad = jnp.pad(ef, ((0, _E_PAD - _E), (0, 0)))
    src3 = jnp.pad(edge_index[0].astype(jnp.int32),
                   (0, _E_PAD - _E)).reshape(2, _NW, _NCHUNK_H, _CHUNK)
    dst3 = jnp.pad(edge_index[1].astype(jnp.int32), (0, _E_PAD - _E),
                   constant_values=_N).reshape(2, _NW, _NCHUNK_H, _CHUNK)
    zeros_sp = jnp.zeros((_N_SP, _D), f32)

    cur = _node_feat(pos_undirected, pos_directed, nf, sd,
                     node_freq_emb, lin0_W, lin0_b.reshape(1, -1))

    a_e = _edge_a(ef_pad, edge_freq_emb, en_W1[0:32], en_W1[32:33].reshape(1, -1),
                  en_b1.reshape(1, -1))
    w2r = en_W2.reshape(_D, _D, _D).reshape(_D * _D, _D).astype(jnp.bfloat16)
    bmat = en_b2.reshape(_D, _D)

    wt = gru_Wih.T
    ht = gru_Whh.T
    gw = [wt[:, 0:32], wt[:, 32:64], wt[:, 64:96],
          ht[:, 0:32], ht[:, 32:64], ht[:, 64:96],
          gru_bih[0:32].reshape(1, -1), gru_bih[32:64].reshape(1, -1),
          gru_bih[64:96].reshape(1, -1),
          gru_bhh[0:32].reshape(1, -1), gru_bhh[32:64].reshape(1, -1),
          gru_bhh[64:96].reshape(1, -1)]
    cb = conv_bias.reshape(1, -1)

    a_h = [a_e[:_E_HALF], a_e[_E_HALF:]]
    for _ in range(_STEPS):
        partials = []
        for hh in range(2):
            xs = _sc_gather_h(cur, src3[hh])
            msg = _msg(xs, a_h[hh], w2r, bmat, _E_HALF)
            partials.append(_sc_scatter_h(msg, dst3[hh], zeros_sp))
        cur = _gru(partials[0], partials[1], cur, cb, gw)

    feat = cur
    featT = feat.T
    gidT = graph_ids.astype(f32).reshape(1, _N)
    w0t = lstm_Wih0.T
    w1t = lstm_Wih12[0].T
    w2t = lstm_Wih12[1].T
    hts = [lstm_Whh[l].T for l in range(3)]
    bs = [(lstm_bih[l] + lstm_bhh[l]).reshape(1, -1) for l in range(3)]

    def gate(mats, g):
        return [m[:, 32 * g:32 * (g + 1)] for m in mats]

    ihs = [w0t, w1t, w2t]
    lstm_parts = []
    for g in range(4):
        lstm_parts.extend(gate(ihs, g))
    for g in range(4):
        lstm_parts.extend(gate(hts, g))
    for g in range(4):
        lstm_parts.extend(gate(bs, g))

    ro = [ro_W1, ro_b1.reshape(1, -1), ro_W2, ro_b2.reshape(1, -1)]
    res = _s2s(feat, featT, gidT, lstm_parts, ro)
    return res


# speedup vs baseline: 1.0251x; 1.0251x over previous
import functools

import jax
import jax.numpy as jnp
from jax import lax
from jax.experimental import pallas as pl
from jax.experimental.pallas import tpu as pltpu
from jax.experimental.pallas import tpu_sc as plsc

f32 = jnp.float32

_N = 10000
_E = 160000
_G = 32
_D = 32
_STEPS = 6
_S2S_ITERS = 6

_NC = 2
_NS = 16
_NW = _NC * _NS
_CHUNK = 128
_NCHUNK = 40
_W_EDGES = _NCHUNK * _CHUNK
_E_PAD = _NW * _W_EDGES
_N_SP = 10016
_ROWS_PER_SUB = _N_SP // _NS

_BLK_N = 2000
_BLK_E = 2048


def _sc_mesh():
    return plsc.VectorSubcoreMesh(core_axis_name="c", subcore_axis_name="s")


_NBUF = 8
_NGRP = _NCHUNK // _NBUF


@functools.partial(
    pl.kernel,
    mesh=_sc_mesh(),
    compiler_params=pltpu.CompilerParams(use_tc_tiling_on_sc=False),
    out_type=jax.ShapeDtypeStruct((_E_PAD, _D), f32),
    scratch_types=[
        pltpu.VMEM((_NCHUNK, _CHUNK), jnp.int32),
        pltpu.VMEM((_NBUF, _CHUNK, _D), f32),
        pltpu.VMEM_SHARED((_N, _D), f32),
        pltpu.SemaphoreType.DMA,
        pltpu.SemaphoreType.DMA,
    ],
)
def _sc_gather(x_hbm, src_hbm, out_hbm, idx_v, rows_v, x_sh, sem_g, sem_o):
    c = lax.axis_index("c")
    s = lax.axis_index("s")
    wid = c * _NS + s
    rows_stage = _N // _NS
    pltpu.sync_copy(x_hbm.at[pl.ds(s * rows_stage, rows_stage)],
                    x_sh.at[pl.ds(s * rows_stage, rows_stage)])
    pltpu.sync_copy(src_hbm.at[wid], idx_v)
    plsc.subcore_barrier()
    base = wid * _W_EDGES

    def group(g, carry):
        @pl.when(g > 0)
        def _drain_prev_out():
            for b in range(_NBUF):
                off = base + ((g - 1) * _NBUF + b) * _CHUNK
                pltpu.make_async_copy(
                    rows_v.at[b], out_hbm.at[pl.ds(off, _CHUNK)], sem_o).wait()
        for b in range(_NBUF):
            pltpu.async_copy(x_sh.at[idx_v.at[g * _NBUF + b]], rows_v.at[b], sem_g)
        for b in range(_NBUF):
            pltpu.make_async_copy(
                x_sh.at[idx_v.at[g * _NBUF + b]], rows_v.at[b], sem_g).wait()
        for b in range(_NBUF):
            off = base + (g * _NBUF + b) * _CHUNK
            pltpu.async_copy(rows_v.at[b], out_hbm.at[pl.ds(off, _CHUNK)], sem_o)
        return carry

    lax.fori_loop(0, _NGRP, group, 0)
    for b in range(_NBUF):
        off = base + ((_NGRP - 1) * _NBUF + b) * _CHUNK
        pltpu.make_async_copy(
            rows_v.at[b], out_hbm.at[pl.ds(off, _CHUNK)], sem_o).wait()


@functools.partial(
    pl.kernel,
    mesh=_sc_mesh(),
    compiler_params=pltpu.CompilerParams(use_tc_tiling_on_sc=False),
    out_type=jax.ShapeDtypeStruct((_NC, _N_SP, _D), f32),
    scratch_types=[
        pltpu.VMEM((_NCHUNK, _CHUNK), jnp.int32),
        pltpu.VMEM((_NBUF, _CHUNK, _D), f32),
        pltpu.VMEM_SHARED((_N_SP, _D), f32),
        pltpu.SemaphoreType.DMA,
        pltpu.SemaphoreType.DMA,
    ],
)
def _sc_scatter(msg_hbm, dst_hbm, zero_hbm, out_hbm, idx_v, rows_v, acc_sh,
                sem_l, sem_s):
    c = lax.axis_index("c")
    s = lax.axis_index("s")
    wid = c * _NS + s

    @pl.when(s == 0)
    def _init():
        pltpu.sync_copy(zero_hbm, acc_sh)

    plsc.subcore_barrier()
    pltpu.sync_copy(dst_hbm.at[wid], idx_v)
    base = wid * _W_EDGES

    def group(g, carry):
        @pl.when(g > 0)
        def _drain_prev_adds():
            for b in range(_NBUF):
                j = (g - 1) * _NBUF + b
                pltpu.make_async_copy(
                    rows_v.at[b], acc_sh.at[idx_v.at[j]], sem_s).wait()
        for b in range(_NBUF):
            j = g * _NBUF + b
            pltpu.async_copy(
                msg_hbm.at[pl.ds(base + j * _CHUNK, _CHUNK)], rows_v.at[b], sem_l)
        for b in range(_NBUF):
            j = g * _NBUF + b
            pltpu.make_async_copy(
                msg_hbm.at[pl.ds(base + j * _CHUNK, _CHUNK)], rows_v.at[b],
                sem_l).wait()
        for b in range(_NBUF):
            j = g * _NBUF + b
            pltpu.async_copy(rows_v.at[b], acc_sh.at[idx_v.at[j]], sem_s, add=True)
        return carry

    lax.fori_loop(0, _NGRP, group, 0)
    for b in range(_NBUF):
        j = (_NGRP - 1) * _NBUF + b
        pltpu.make_async_copy(rows_v.at[b], acc_sh.at[idx_v.at[j]], sem_s).wait()
    plsc.subcore_barrier()
    r0 = s * _ROWS_PER_SUB
    pltpu.sync_copy(acc_sh.at[pl.ds(r0, _ROWS_PER_SUB)],
                    out_hbm.at[c, pl.ds(r0, _ROWS_PER_SUB)])


def _full(shape):
    return pl.BlockSpec(shape, lambda *_: (0,) * len(shape))


def _node_feat_body(pu, pd, nf, sd, emb, w, b, o_ref):
    nfv = nf[...]
    oh = (nfv == lax.broadcasted_iota(jnp.int32, (_BLK_N, 9), 1).astype(f32)).astype(f32)
    nf_emb = jnp.dot(oh, emb[...], preferred_element_type=f32)
    wv = w[...]
    x = (jnp.dot(pu[...], wv[0:16], preferred_element_type=f32)
         + jnp.dot(pd[...], wv[16:32], preferred_element_type=f32)
         + jnp.dot(nf_emb, wv[32:64], preferred_element_type=f32)
         + sd[...] * wv[64:65] + (nfv / 8.0) * wv[65:66] + b[...])
    o_ref[...] = jnp.maximum(x, 0.0)


def _node_feat(pu, pd, nf, sd, emb, w, b):
    grid = (_N // _BLK_N,)
    return pl.pallas_call(
        _node_feat_body,
        grid=grid,
        in_specs=[
            pl.BlockSpec((_BLK_N, 16), lambda i: (i, 0)),
            pl.BlockSpec((_BLK_N, 16), lambda i: (i, 0)),
            pl.BlockSpec((_BLK_N, 1), lambda i: (i, 0)),
            pl.BlockSpec((_BLK_N, 1), lambda i: (i, 0)),
            _full((9, 32)), _full((66, 32)), _full((1, 32)),
        ],
        out_specs=pl.BlockSpec((_BLK_N, _D), lambda i: (i, 0)),
        out_shape=jax.ShapeDtypeStruct((_N, _D), f32),
    )(pu, pd, nf, sd, emb, w, b)


def _edge_a_body(ef, emb, w1a, w1l, b1, o_ref):
    efv = ef[...]
    oh = (efv == lax.broadcasted_iota(jnp.int32, (_BLK_E, 9), 1).astype(f32)).astype(f32)
    ef_emb = jnp.dot(oh, emb[...], preferred_element_type=f32)
    o_ref[...] = jnp.maximum(
        jnp.dot(ef_emb, w1a[...], preferred_element_type=f32)
        + (efv / 8.0) * w1l[...] + b1[...], 0.0)


def _edge_a(ef, emb, w1a, w1l, b1):
    grid = (_E_PAD // _BLK_E,)
    return pl.pallas_call(
        _edge_a_body,
        grid=grid,
        in_specs=[
            pl.BlockSpec((_BLK_E, 1), lambda i: (i, 0)),
            _full((9, 32)), _full((32, 32)), _full((1, 32)), _full((1, 32)),
        ],
        out_specs=pl.BlockSpec((_BLK_E, 32), lambda i: (i, 0)),
        out_shape=jax.ShapeDtypeStruct((_E_PAD, 32), f32),
    )(ef, emb, w1a, w1l, b1)


def _msg_body(xs, a, w2r, bmat, o_ref):
    bf = jnp.bfloat16
    xv = xs[...]
    av = a[...]
    ci = lax.broadcasted_iota(jnp.int32, (32, 1024), 1)
    rk = lax.broadcasted_iota(jnp.int32, (32, 1024), 0)
    rep = (ci // 32 == rk).astype(bf)
    arep = jnp.dot(av.astype(bf), rep, preferred_element_type=f32)
    xtile = pltpu.repeat(xv, 32, axis=1)
    u = (arep * xtile).astype(bf)
    o_ref[...] = (jnp.dot(u, w2r[...], preferred_element_type=f32)
                  + jnp.dot(xv, bmat[...], preferred_element_type=f32))


def _msg(xs, a, w2r, bmat):
    grid = (_E_PAD // _BLK_E,)
    return pl.pallas_call(
        _msg_body,
        grid=grid,
        in_specs=[
            pl.BlockSpec((_BLK_E, _D), lambda i: (i, 0)),
            pl.BlockSpec((_BLK_E, _D), lambda i: (i, 0)),
            _full((1024, 32)), _full((32, 32)),
        ],
        out_specs=pl.BlockSpec((_BLK_E, _D), lambda i: (i, 0)),
        out_shape=jax.ShapeDtypeStruct((_E_PAD, _D), f32),
    )(xs, a, w2r, bmat)


def _gru_body(p0, p1, h, cb, wir, wiz, win, whr, whz, whn,
              bir, biz, bin_, bhr, bhz, bhn, o_ref):
    agg = (p0[...] + p1[...])[0]
    m = jnp.maximum(agg + cb[...], 0.0)
    hv = h[...]
    dot = lambda a, b: jnp.dot(a, b, preferred_element_type=f32)
    r = jax.nn.sigmoid(dot(m, wir[...]) + bir[...] + dot(hv, whr[...]) + bhr[...])
    z = jax.nn.sigmoid(dot(m, wiz[...]) + biz[...] + dot(hv, whz[...]) + bhz[...])
    nn_ = jnp.tanh(dot(m, win[...]) + bin_[...] + r * (dot(hv, whn[...]) + bhn[...]))
    o_ref[...] = (1.0 - z) * nn_ + z * hv


def _gru(parts, h, cb, ws):
    grid = (_N // _BLK_N,)
    specs = [pl.BlockSpec((1, _BLK_N, _D), lambda i: (0, i, 0)),
             pl.BlockSpec((1, _BLK_N, _D), lambda i: (1, i, 0)),
             pl.BlockSpec((_BLK_N, _D), lambda i: (i, 0))]
    specs += [_full((1, 32))]
    specs += [_full((32, 32))] * 6
    specs += [_full((1, 32))] * 6
    return pl.pallas_call(
        _gru_body,
        grid=grid,
        in_specs=specs,
        out_specs=pl.BlockSpec((_BLK_N, _D), lambda i: (i, 0)),
        out_shape=jax.ShapeDtypeStruct((_N, _D), f32),
    )(parts, parts, h, cb, *ws)


def _s2s_body(feat, featT, gidT, *refs):
    out_ref = refs[-1]
    flat = refs[:-1]
    wih_i, wih_f, wih_g, wih_o = [flat[3 * g:3 * g + 3] for g in range(4)]
    whh_i, whh_f, whh_g, whh_o = [flat[12 + 3 * g:12 + 3 * g + 3] for g in range(4)]
    b_i, b_f, b_g, b_o = [flat[24 + 3 * g:24 + 3 * g + 3] for g in range(4)]
    row1, rob1, row2, rob2 = flat[36:40]

    featv = feat[...]
    featTv = featT[...]
    mT = gidT[...] == lax.broadcasted_iota(jnp.int32, (_G, _N), 0).astype(f32)

    dot = lambda a, b: jnp.dot(a, b, preferred_element_type=f32)
    hs = [jnp.zeros((_G, _D), f32) for _ in range(3)]
    cs = [jnp.zeros((_G, _D), f32) for _ in range(3)]
    q_star = jnp.zeros((_G, 2 * _D), f32)
    for _ in range(_S2S_ITERS):
        x_in = q_star
        for l in range(3):
            pre_i = dot(x_in, wih_i[l][...]) + dot(hs[l], whh_i[l][...]) + b_i[l][...]
            pre_f = dot(x_in, wih_f[l][...]) + dot(hs[l], whh_f[l][...]) + b_f[l][...]
            pre_g = dot(x_in, wih_g[l][...]) + dot(hs[l], whh_g[l][...]) + b_g[l][...]
            pre_o = dot(x_in, wih_o[l][...]) + dot(hs[l], whh_o[l][...]) + b_o[l][...]
            ig = jax.nn.sigmoid(pre_i)
            fg = jax.nn.sigmoid(pre_f)
            gg = jnp.tanh(pre_g)
            og = jax.nn.sigmoid(pre_o)
            cs[l] = fg * cs[l] + ig * gg
            hs[l] = og * jnp.tanh(cs[l])
            x_in = hs[l]
        q = x_in
        e_allT = dot(q, featTv)
        emaxT = jnp.max(jnp.where(mT, e_allT, -1e30), axis=1, keepdims=True)
        exT = jnp.exp(jnp.where(mT, e_allT - emaxT, -1e30))
        esumT = jnp.sum(exT, axis=1, keepdims=True)
        alphaT = exT / jnp.maximum(esumT, 1e-30)
        readout = dot(alphaT, featv)
        q_star = jnp.concatenate([q, readout], axis=-1)
    hid = jnp.maximum(dot(q_star, row1[...]) + rob1[...], 0.0)
    out_ref[...] = dot(hid, row2[...]) + rob2[...]


def _s2s(feat, featT, gidT, lstm_parts, ro):
    args = [feat, featT, gidT] + list(lstm_parts) + list(ro)
    in_specs = [_full(a.shape) for a in args]
    return pl.pallas_call(
        _s2s_body,
        in_specs=in_specs,
        out_specs=_full((_G, 32)),
        out_shape=jax.ShapeDtypeStruct((_G, 32), f32),
        grid=(1,),
    )(*args)


def kernel(pos_undirected, pos_directed, nfreq, seed, efreq, edge_index, graph_ids,
           node_freq_emb, edge_freq_emb, lin0_W, lin0_b, en_W1, en_b1, en_W2, en_b2,
           conv_bias, gru_Wih, gru_Whh, gru_bih, gru_bhh,
           lstm_Wih0, lstm_Wih12, lstm_Whh, lstm_bih, lstm_bhh,
           ro_W1, ro_b1, ro_W2, ro_b2):
    nf = jnp.clip(nfreq, 0, 8).astype(f32)[:, None]
    sd = seed.astype(f32)[:, None]
    ef = jnp.clip(efreq, 0, 8).astype(f32)[:, None]
    ef_pad = jnp.pad(ef, ((0, _E_PAD - _E), (0, 0)))
    src3 = jnp.pad(edge_index[0].astype(jnp.int32),
                   (0, _E_PAD - _E)).reshape(_NW, _NCHUNK, _CHUNK)
    dst3 = jnp.pad(edge_index[1].astype(jnp.int32), (0, _E_PAD - _E),
                   constant_values=_N).reshape(_NW, _NCHUNK, _CHUNK)
    zeros_sp = jnp.zeros((_N_SP, _D), f32)

    cur = _node_feat(pos_undirected, pos_directed, nf, sd,
                     node_freq_emb, lin0_W, lin0_b.reshape(1, -1))

    a_e = _edge_a(ef_pad, edge_freq_emb, en_W1[0:32], en_W1[32:33].reshape(1, -1),
                  en_b1.reshape(1, -1))
    w2r = en_W2.reshape(_D, _D, _D).reshape(_D * _D, _D).astype(jnp.bfloat16)
    bmat = en_b2.reshape(_D, _D)

    wt = gru_Wih.T
    ht = gru_Whh.T
    gw = [wt[:, 0:32], wt[:, 32:64], wt[:, 64:96],
          ht[:, 0:32], ht[:, 32:64], ht[:, 64:96],
          gru_bih[0:32].reshape(1, -1), gru_bih[32:64].reshape(1, -1),
          gru_bih[64:96].reshape(1, -1),
          gru_bhh[0:32].reshape(1, -1), gru_bhh[32:64].reshape(1, -1),
          gru_bhh[64:96].reshape(1, -1)]
    cb = conv_bias.reshape(1, -1)

    for _ in range(_STEPS):
        xs = _sc_gather(cur, src3)
        msg = _msg(xs, a_e, w2r, bmat)
        parts = _sc_scatter(msg, dst3, zeros_sp)
        cur = _gru(parts, cur, cb, gw)

    feat = cur
    featT = feat.T
    gidT = graph_ids.astype(f32).reshape(1, _N)
    w0t = lstm_Wih0.T
    w1t = lstm_Wih12[0].T
    w2t = lstm_Wih12[1].T
    hts = [lstm_Whh[l].T for l in range(3)]
    bs = [(lstm_bih[l] + lstm_bhh[l]).reshape(1, -1) for l in range(3)]

    def gate(mats, g):
        return [m[:, 32 * g:32 * (g + 1)] for m in mats]

    ihs = [w0t, w1t, w2t]
    lstm_parts = []
    for g in range(4):
        lstm_parts.extend(gate(ihs, g))
    for g in range(4):
        lstm_parts.extend(gate(hts, g))
    for g in range(4):
        lstm_parts.extend(gate(bs, g))

    ro = [ro_W1, ro_b1.reshape(1, -1), ro_W2, ro_b2.reshape(1, -1)]
    res = _s2s(feat, featT, gidT, lstm_parts, ro)
    return res
